# baseline (speedup 1.0000x reference)
import functools

import jax
import jax.numpy as jnp
from jax import lax
from jax.experimental import pallas as pl
from jax.experimental.pallas import tpu as pltpu
from jax.experimental.pallas import tpu_sc as plsc

NC = 2
NS = 16
NW = NC * NS
LANES = 16
CH = 128
HID = 128
NP = 10240
N_UI = 10240
N_UU = 6144
NK_UI = 80
NK_UU = 40
RB = 1000


def _mesh():
    return plsc.VectorSubcoreMesh(core_axis_name="c", subcore_axis_name="s")


@functools.lru_cache(maxsize=None)
def _make_degrees(nk):
    def body(src3, dst3, out, srcv, dstv, hist_s, hist_d):
        cid = lax.axis_index("c")
        sid = lax.axis_index("s")
        wid = sid * NC + cid

        def fz(r, c):
            hist_s[pl.ds(r * LANES, LANES)] = jnp.zeros((LANES,), jnp.float32)
            hist_d[pl.ds(r * LANES, LANES)] = jnp.zeros((LANES,), jnp.float32)
            return c
        lax.fori_loop(0, NP // LANES, fz, 0)

        pltpu.sync_copy(src3.at[wid], srcv)
        pltpu.sync_copy(dst3.at[wid], dstv)

        ones = jnp.ones((LANES,), jnp.float32)

        def step(r, c):
            for j in range(CH // LANES):
                plsc.addupdate_scatter(
                    hist_s, [srcv[r, pl.ds(j * LANES, LANES)]], ones)
                plsc.addupdate_scatter(
                    hist_d, [dstv[r, pl.ds(j * LANES, LANES)]], ones)
            return c
        lax.fori_loop(0, nk, step, 0)

        pltpu.sync_copy(hist_s, out.at[wid, 0])
        pltpu.sync_copy(hist_d, out.at[wid, 1])

    return pl.kernel(
        body,
        mesh=_mesh(),
        out_type=jax.ShapeDtypeStruct((NW, 2, NP), jnp.float32),
        compiler_params=pltpu.CompilerParams(needs_layout_passes=False),
        scratch_types=[
            pltpu.VMEM((nk, CH), jnp.int32),
            pltpu.VMEM((nk, CH), jnp.int32),
            pltpu.VMEM((NP,), jnp.float32),
            pltpu.VMEM((NP,), jnp.float32),
        ],
    )


@functools.lru_cache(maxsize=None)
def _make_scales():
    B = 1024
    nb = NP // B

    def body(c_ref, o_ref, acc_ref):
        t = pl.program_id(2)

        @pl.when(t == 0)
        def _():
            acc_ref[...] = jnp.zeros_like(acc_ref)

        acc_ref[...] += c_ref[0, 0]

        @pl.when(t == NW - 1)
        def _():
            o_ref[...] = lax.rsqrt(jnp.maximum(acc_ref[...], 1.0))[None]

    return pl.pallas_call(
        body,
        grid=(2, nb, NW),
        in_specs=[
            pl.BlockSpec((1, 1, B, 1), lambda d, j, t: (t, d, j, 0)),
        ],
        out_specs=pl.BlockSpec((1, B, 1), lambda d, j, t: (d, j, 0)),
        out_shape=jax.ShapeDtypeStruct((2, NP, 1), jnp.float32),
        scratch_shapes=[pltpu.VMEM((B, 1), jnp.float32)],
    )


def _zero_chunks(rpt):
    out, left = [], rpt
    while left > 0:
        out.append(min(left, CH))
        left -= min(left, CH)
    return out


@functools.lru_cache(maxsize=None)
def _make_aggregate():
    rpt_ui = N_UI // NS
    rpt_uu = N_UU // NS

    def body(nodef_ui, src_ui, dst_ui, nodef_uu, src_uu, dst_uu,
             out_ui, out_uu,
             srcv_ui, dstv_ui, srcv_uu, dstv_uu,
             rows, acc):
        rows0 = rows.at[pl.ds(0, CH)]
        cid = lax.axis_index("c")
        sid = lax.axis_index("s")
        wid = sid * NC + cid

        for src3, dst3, srcv, dstv, nk in (
                (src_ui, dst_ui, srcv_ui, dstv_ui, NK_UI),
                (src_uu, dst_uu, srcv_uu, dstv_uu, NK_UU)):
            def ld(q, c, src3=src3, dst3=dst3, srcv=srcv, dstv=dstv):
                pltpu.sync_copy(src3.at[wid, pl.ds(q * 8, 8)],
                                srcv.at[pl.ds(q * 8, 8)])
                pltpu.sync_copy(dst3.at[wid, pl.ds(q * 8, 8)],
                                dstv.at[pl.ds(q * 8, 8)])
                return c
            lax.fori_loop(0, nk // 8, ld, 0)

        def phase(nodef, srcv, dstv, out, rpt, npairs):
            def fz(r, c):
                for j in range(HID // LANES):
                    rows[r, pl.ds(j * LANES, LANES)] = jnp.zeros(
                        (LANES,), jnp.float32)
                return c
            lax.fori_loop(0, CH, fz, 0)

            def zc(z, c):
                pltpu.sync_copy(rows.at[pl.ds(0, CH)],
                                acc.at[pl.ds(sid * rpt + z * CH, CH)])
                return c
            lax.fori_loop(0, rpt // CH, zc, 0)
            plsc.subcore_barrier()

            def step(k, c):
                pltpu.sync_copy(nodef.at[srcv.at[k]], rows0)
                pltpu.sync_copy(rows0, acc.at[dstv.at[k]], add=True)
                return c
            lax.fori_loop(0, 2 * npairs, step, 0)

            plsc.subcore_barrier()
            pltpu.sync_copy(acc.at[pl.ds(sid * rpt, rpt)],
                            out.at[cid, pl.ds(sid * rpt, rpt)])
            plsc.subcore_barrier()

        phase(nodef_ui, srcv_ui, dstv_ui, out_ui, rpt_ui, NK_UI // 2)
        phase(nodef_uu, srcv_uu, dstv_uu, out_uu, rpt_uu, NK_UU // 2)

    return pl.kernel(
        body,
        mesh=_mesh(),
        out_type=(jax.ShapeDtypeStruct((NC, N_UI, HID), jnp.float32),
                  jax.ShapeDtypeStruct((NC, N_UU, HID), jnp.float32)),
        compiler_params=pltpu.CompilerParams(needs_layout_passes=False),
        scratch_types=[
            pltpu.VMEM((NK_UI, CH), jnp.int32),
            pltpu.VMEM((NK_UI, CH), jnp.int32),
            pltpu.VMEM((NK_UU, CH), jnp.int32),
            pltpu.VMEM((NK_UU, CH), jnp.int32),
            pltpu.VMEM((CH, HID), jnp.float32),
            pltpu.VMEM_SHARED((N_UI, HID), jnp.float32),
        ],
    )


@functools.lru_cache(maxsize=None)
def _make_matmul_ui():
    def body(x_ref, wu_ref, wv_ref, s_ref, o_ref):
        i = pl.program_id(0)
        w = jnp.where(i < 5, wu_ref[...], wv_ref[...])
        o_ref[...] = jnp.dot(x_ref[...], w,
                             preferred_element_type=jnp.float32) * s_ref[0]

    return pl.pallas_call(
        body,
        grid=(10,),
        in_specs=[
            pl.BlockSpec((RB, HID), lambda i: (i, 0)),
            pl.BlockSpec((HID, HID), lambda i: (0, 0)),
            pl.BlockSpec((HID, HID), lambda i: (0, 0)),
            pl.BlockSpec((1, RB, 1), lambda i: (0, i, 0)),
        ],
        out_specs=pl.BlockSpec((RB, HID), lambda i: (i, 0)),
        out_shape=jax.ShapeDtypeStruct((N_UI, HID), jnp.float32),
    )


@functools.lru_cache(maxsize=None)
def _make_matmul_uu():
    def body(x_ref, w_ref, s_ref, o_ref):
        o_ref[...] = jnp.dot(x_ref[...], w_ref[...],
                             preferred_element_type=jnp.float32) * s_ref[0]

    return pl.pallas_call(
        body,
        grid=(5,),
        in_specs=[
            pl.BlockSpec((RB, HID), lambda i: (i, 0)),
            pl.BlockSpec((HID, HID), lambda i: (0, 0)),
            pl.BlockSpec((1, RB, 1), lambda i: (0, i, 0)),
        ],
        out_specs=pl.BlockSpec((RB, HID), lambda i: (i, 0)),
        out_shape=jax.ShapeDtypeStruct((N_UU, HID), jnp.float32),
    )


@functools.lru_cache(maxsize=None)
def _make_post(n_pad, n_real):
    nb = n_real // RB

    def body(a0, a1, s_ref, c_ref, emb_ref, out_ref):
        x = (a0[0] + a1[0]) * s_ref[0]
        e = jnp.where(x >= 0, x, 0.5 * x)
        emb_ref[...] = e
        n = jnp.sqrt(jnp.sum(e * e, axis=1, keepdims=True))
        out_ref[...] = c_ref[...] + e / jnp.maximum(n, 1e-12)

    return pl.pallas_call(
        body,
        grid=(nb,),
        in_specs=[
            pl.BlockSpec((1, RB, HID), lambda i: (0, i, 0)),
            pl.BlockSpec((1, RB, HID), lambda i: (1, i, 0)),
            pl.BlockSpec((1, RB, 1), lambda i: (1, i, 0)),
            pl.BlockSpec((RB, HID), lambda i: (i, 0)),
        ],
        out_specs=[
            pl.BlockSpec((RB, HID), lambda i: (i, 0)),
            pl.BlockSpec((RB, HID), lambda i: (i, 0)),
        ],
        out_shape=[
            jax.ShapeDtypeStruct((n_pad, HID), jnp.float32),
            jax.ShapeDtypeStruct((n_real, HID), jnp.float32),
        ],
    )


@functools.lru_cache(maxsize=None)
def _make_init_ui():
    def body(xu_ref, xv_ref, emb_ref, c_ref):
        i = pl.program_id(0)
        v = jnp.where(i < 5, xu_ref[...], xv_ref[...])
        emb_ref[...] = v
        c_ref[...] = v

    return pl.pallas_call(
        body,
        grid=(10,),
        in_specs=[
            pl.BlockSpec((RB, HID), lambda i: (jnp.where(i < 5, i, 0), 0)),
            pl.BlockSpec((RB, HID), lambda i: (jnp.where(i < 5, 0, i - 5), 0)),
        ],
        out_specs=[
            pl.BlockSpec((RB, HID), lambda i: (i, 0)),
            pl.BlockSpec((RB, HID), lambda i: (i, 0)),
        ],
        out_shape=[
            jax.ShapeDtypeStruct((N_UI, HID), jnp.float32),
            jax.ShapeDtypeStruct((10000, HID), jnp.float32),
        ],
    )


@functools.lru_cache(maxsize=None)
def _make_init_uu():
    def body(x_ref, o_ref):
        o_ref[...] = x_ref[...]

    return pl.pallas_call(
        body,
        grid=(5,),
        in_specs=[pl.BlockSpec((RB, HID), lambda i: (i, 0))],
        out_specs=pl.BlockSpec((RB, HID), lambda i: (i, 0)),
        out_shape=jax.ShapeDtypeStruct((N_UU, HID), jnp.float32),
    )


def _prep_edges(ei, nk, pad_idx):
    e = ei.shape[1]
    tot = NW * nk * CH
    ei = ei.astype(jnp.int32)
    pad = jnp.full((tot - e,), pad_idx, jnp.int32)
    src = jnp.concatenate([ei[0], pad]).reshape(NW, nk, CH)
    dst = jnp.concatenate([ei[1], pad]).reshape(NW, nk, CH)
    return src, dst


def kernel(user_embeddings, item_embeddings, ui_edge_index, uu_edge_index,
           ui_u_w, ui_v_w, uu_u_w):
    ui_src, ui_dst = _prep_edges(ui_edge_index, NK_UI, 10000)
    uu_src, uu_dst = _prep_edges(uu_edge_index, NK_UU, 5000)

    scales = _make_scales()
    deg_ui = _make_degrees(NK_UI)(ui_src, ui_dst)
    deg_uu = _make_degrees(NK_UU)(uu_src, uu_dst)
    sc_ui = scales(deg_ui.reshape(NW, 2, NP, 1))
    sc_uu = scales(deg_uu.reshape(NW, 2, NP, 1))

    agg = _make_aggregate()
    mm_ui = _make_matmul_ui()
    mm_uu = _make_matmul_uu()
    post_ui = _make_post(N_UI, 10000)
    post_uu = _make_post(N_UU, 5000)

    emb_ui0, carry_ui0 = _make_init_ui()(user_embeddings, item_embeddings)
    emb_uu0 = _make_init_uu()(user_embeddings)

    def layer(c, ws):
        emb_ui, carry_ui, emb_uu, carry_uu = c
        wu, wv, wuu = ws
        nodef_ui = mm_ui(emb_ui, wu, wv, sc_ui)
        nodef_uu = mm_uu(emb_uu, wuu, sc_uu)
        parts_ui, parts_uu = agg(nodef_ui, ui_src, ui_dst,
                                 nodef_uu, uu_src, uu_dst)
        emb_ui, carry_ui = post_ui(parts_ui, parts_ui, sc_ui, carry_ui)
        emb_uu, carry_uu = post_uu(parts_uu, parts_uu, sc_uu, carry_uu)
        return (emb_ui, carry_ui, emb_uu, carry_uu), None

    c0 = (emb_ui0, carry_ui0, emb_uu0, user_embeddings)
    (_, carry_ui, _, carry_uu), _ = lax.scan(
        layer, c0, (ui_u_w, ui_v_w, uu_u_w))

    return carry_ui, carry_uu


# speedup vs baseline: 1.1100x; 1.1100x over previous
import functools

import jax
import jax.numpy as jnp
from jax import lax
from jax.experimental import pallas as pl
from jax.experimental.pallas import tpu as pltpu
from jax.experimental.pallas import tpu_sc as plsc

NC = 2
NS = 16
NW = NC * NS
LANES = 16
CH = 128
HID = 128
NP = 10240
N_UI = 10112
N_UU = 5120
NK_UI = 80
NK_UU = 40
RB = 1000


def _mesh():
    return plsc.VectorSubcoreMesh(core_axis_name="c", subcore_axis_name="s")


@functools.lru_cache(maxsize=None)
def _make_degrees(nk):
    def body(src3, dst3, out, srcv, dstv, hist_s, hist_d):
        cid = lax.axis_index("c")
        sid = lax.axis_index("s")
        wid = sid * NC + cid

        def fz(r, c):
            hist_s[pl.ds(r * LANES, LANES)] = jnp.zeros((LANES,), jnp.float32)
            hist_d[pl.ds(r * LANES, LANES)] = jnp.zeros((LANES,), jnp.float32)
            return c
        lax.fori_loop(0, NP // LANES, fz, 0)

        pltpu.sync_copy(src3.at[wid], srcv)
        pltpu.sync_copy(dst3.at[wid], dstv)

        ones = jnp.ones((LANES,), jnp.float32)

        def step(r, c):
            for j in range(CH // LANES):
                plsc.addupdate_scatter(
                    hist_s, [srcv[r, pl.ds(j * LANES, LANES)]], ones)
                plsc.addupdate_scatter(
                    hist_d, [dstv[r, pl.ds(j * LANES, LANES)]], ones)
            return c
        lax.fori_loop(0, nk, step, 0)

        pltpu.sync_copy(hist_s, out.at[wid, 0])
        pltpu.sync_copy(hist_d, out.at[wid, 1])

    return pl.kernel(
        body,
        mesh=_mesh(),
        out_type=jax.ShapeDtypeStruct((NW, 2, NP), jnp.float32),
        compiler_params=pltpu.CompilerParams(needs_layout_passes=False),
        scratch_types=[
            pltpu.VMEM((nk, CH), jnp.int32),
            pltpu.VMEM((nk, CH), jnp.int32),
            pltpu.VMEM((NP,), jnp.float32),
            pltpu.VMEM((NP,), jnp.float32),
        ],
    )


@functools.lru_cache(maxsize=None)
def _make_scales():
    B = 1024
    nb = NP // B

    def body(c_ref, o_ref, acc_ref):
        t = pl.program_id(2)

        @pl.when(t == 0)
        def _():
            acc_ref[...] = jnp.zeros_like(acc_ref)

        acc_ref[...] += c_ref[0, 0]

        @pl.when(t == NW - 1)
        def _():
            o_ref[...] = lax.rsqrt(jnp.maximum(acc_ref[...], 1.0))[None]

    return pl.pallas_call(
        body,
        grid=(2, nb, NW),
        in_specs=[
            pl.BlockSpec((1, 1, B, 1), lambda d, j, t: (t, d, j, 0)),
        ],
        out_specs=pl.BlockSpec((1, B, 1), lambda d, j, t: (d, j, 0)),
        out_shape=jax.ShapeDtypeStruct((2, NP, 1), jnp.float32),
        scratch_shapes=[pltpu.VMEM((B, 1), jnp.float32)],
    )


def _zero_chunks(rpt):
    out, left = [], rpt
    while left > 0:
        out.append(min(left, CH))
        left -= min(left, CH)
    return out


@functools.lru_cache(maxsize=None)
def _make_aggregate():
    rpt_ui = N_UI // NS
    rpt_uu = N_UU // NS

    def body(nodef_ui, src_ui, dst_ui, nodef_uu, src_uu, dst_uu,
             out_ui, out_uu,
             srcv_ui, dstv_ui, srcv_uu, dstv_uu,
             rows0, acc):
        cid = lax.axis_index("c")
        sid = lax.axis_index("s")
        wid = sid * NC + cid

        for src3, dst3, srcv, dstv, nk in (
                (src_ui, dst_ui, srcv_ui, dstv_ui, NK_UI),
                (src_uu, dst_uu, srcv_uu, dstv_uu, NK_UU)):
            def ld(q, c, src3=src3, dst3=dst3, srcv=srcv, dstv=dstv):
                pltpu.sync_copy(src3.at[wid, pl.ds(q * 8, 8)],
                                srcv.at[pl.ds(q * 8, 8)])
                pltpu.sync_copy(dst3.at[wid, pl.ds(q * 8, 8)],
                                dstv.at[pl.ds(q * 8, 8)])
                return c
            lax.fori_loop(0, nk // 8, ld, 0)

        def phase(nodef, srcv, dstv, out, rpt, nchunks):
            def fz(r, c):
                for j in range(HID // LANES):
                    rows0[r, pl.ds(j * LANES, LANES)] = jnp.zeros(
                        (LANES,), jnp.float32)
                return c
            lax.fori_loop(0, CH, fz, 0)
            off = 0
            for zc in _zero_chunks(rpt):
                pltpu.sync_copy(rows0.at[pl.ds(0, zc)],
                                acc.at[pl.ds(sid * rpt + off, zc)])
                off += zc
            plsc.subcore_barrier()

            def step(k, c):
                pltpu.sync_copy(nodef.at[srcv.at[k]], rows0)
                pltpu.sync_copy(rows0, acc.at[dstv.at[k]], add=True)
                return c
            lax.fori_loop(0, nchunks, step, 0)

            plsc.subcore_barrier()
            pltpu.sync_copy(acc.at[pl.ds(sid * rpt, rpt)],
                            out.at[cid, pl.ds(sid * rpt, rpt)])
            plsc.subcore_barrier()

        phase(nodef_ui, srcv_ui, dstv_ui, out_ui, rpt_ui, NK_UI)
        phase(nodef_uu, srcv_uu, dstv_uu, out_uu, rpt_uu, NK_UU)

    return pl.kernel(
        body,
        mesh=_mesh(),
        out_type=(jax.ShapeDtypeStruct((NC, N_UI, HID), jnp.float32),
                  jax.ShapeDtypeStruct((NC, N_UU, HID), jnp.float32)),
        compiler_params=pltpu.CompilerParams(needs_layout_passes=False),
        scratch_types=[
            pltpu.VMEM((NK_UI, CH), jnp.int32),
            pltpu.VMEM((NK_UI, CH), jnp.int32),
            pltpu.VMEM((NK_UU, CH), jnp.int32),
            pltpu.VMEM((NK_UU, CH), jnp.int32),
            pltpu.VMEM((CH, HID), jnp.float32),
            pltpu.VMEM_SHARED((N_UI, HID), jnp.float32),
        ],
    )


@functools.lru_cache(maxsize=None)
def _make_matmul_ui():
    def body(x_ref, wu_ref, wv_ref, s_ref, o_ref):
        i = pl.program_id(0)
        w = jnp.where(i < 5, wu_ref[...], wv_ref[...])
        o_ref[...] = jnp.dot(x_ref[...], w,
                             preferred_element_type=jnp.float32) * s_ref[0]

    return pl.pallas_call(
        body,
        grid=(10,),
        in_specs=[
            pl.BlockSpec((RB, HID), lambda i: (i, 0)),
            pl.BlockSpec((HID, HID), lambda i: (0, 0)),
            pl.BlockSpec((HID, HID), lambda i: (0, 0)),
            pl.BlockSpec((1, RB, 1), lambda i: (0, i, 0)),
        ],
        out_specs=pl.BlockSpec((RB, HID), lambda i: (i, 0)),
        out_shape=jax.ShapeDtypeStruct((N_UI, HID), jnp.float32),
    )


@functools.lru_cache(maxsize=None)
def _make_matmul_uu():
    def body(x_ref, w_ref, s_ref, o_ref):
        o_ref[...] = jnp.dot(x_ref[...], w_ref[...],
                             preferred_element_type=jnp.float32) * s_ref[0]

    return pl.pallas_call(
        body,
        grid=(5,),
        in_specs=[
            pl.BlockSpec((RB, HID), lambda i: (i, 0)),
            pl.BlockSpec((HID, HID), lambda i: (0, 0)),
            pl.BlockSpec((1, RB, 1), lambda i: (0, i, 0)),
        ],
        out_specs=pl.BlockSpec((RB, HID), lambda i: (i, 0)),
        out_shape=jax.ShapeDtypeStruct((N_UU, HID), jnp.float32),
    )


@functools.lru_cache(maxsize=None)
def _make_post(n_pad, n_real):
    nb = n_real // RB

    def body(a0, a1, s_ref, c_ref, emb_ref, out_ref):
        x = (a0[0] + a1[0]) * s_ref[0]
        e = jnp.where(x >= 0, x, 0.5 * x)
        emb_ref[...] = e
        n = jnp.sqrt(jnp.sum(e * e, axis=1, keepdims=True))
        out_ref[...] = c_ref[...] + e / jnp.maximum(n, 1e-12)

    return pl.pallas_call(
        body,
        grid=(nb,),
        in_specs=[
            pl.BlockSpec((1, RB, HID), lambda i: (0, i, 0)),
            pl.BlockSpec((1, RB, HID), lambda i: (1, i, 0)),
            pl.BlockSpec((1, RB, 1), lambda i: (1, i, 0)),
            pl.BlockSpec((RB, HID), lambda i: (i, 0)),
        ],
        out_specs=[
            pl.BlockSpec((RB, HID), lambda i: (i, 0)),
            pl.BlockSpec((RB, HID), lambda i: (i, 0)),
        ],
        out_shape=[
            jax.ShapeDtypeStruct((n_pad, HID), jnp.float32),
            jax.ShapeDtypeStruct((n_real, HID), jnp.float32),
        ],
    )


@functools.lru_cache(maxsize=None)
def _make_init_ui():
    def body(xu_ref, xv_ref, emb_ref, c_ref):
        i = pl.program_id(0)
        v = jnp.where(i < 5, xu_ref[...], xv_ref[...])
        emb_ref[...] = v
        c_ref[...] = v

    return pl.pallas_call(
        body,
        grid=(10,),
        in_specs=[
            pl.BlockSpec((RB, HID), lambda i: (jnp.where(i < 5, i, 0), 0)),
            pl.BlockSpec((RB, HID), lambda i: (jnp.where(i < 5, 0, i - 5), 0)),
        ],
        out_specs=[
            pl.BlockSpec((RB, HID), lambda i: (i, 0)),
            pl.BlockSpec((RB, HID), lambda i: (i, 0)),
        ],
        out_shape=[
            jax.ShapeDtypeStruct((N_UI, HID), jnp.float32),
            jax.ShapeDtypeStruct((10000, HID), jnp.float32),
        ],
    )


@functools.lru_cache(maxsize=None)
def _make_init_uu():
    def body(x_ref, o_ref):
        o_ref[...] = x_ref[...]

    return pl.pallas_call(
        body,
        grid=(5,),
        in_specs=[pl.BlockSpec((RB, HID), lambda i: (i, 0))],
        out_specs=pl.BlockSpec((RB, HID), lambda i: (i, 0)),
        out_shape=jax.ShapeDtypeStruct((N_UU, HID), jnp.float32),
    )


def _prep_edges(ei, nk, pad_idx):
    e = ei.shape[1]
    tot = NW * nk * CH
    ei = ei.astype(jnp.int32)
    pad = jnp.full((tot - e,), pad_idx, jnp.int32)
    src = jnp.concatenate([ei[0], pad]).reshape(NW, nk, CH)
    dst = jnp.concatenate([ei[1], pad]).reshape(NW, nk, CH)
    return src, dst


def kernel(user_embeddings, item_embeddings, ui_edge_index, uu_edge_index,
           ui_u_w, ui_v_w, uu_u_w):
    ui_src, ui_dst = _prep_edges(ui_edge_index, NK_UI, 10000)
    uu_src, uu_dst = _prep_edges(uu_edge_index, NK_UU, 5000)

    scales = _make_scales()
    deg_ui = _make_degrees(NK_UI)(ui_src, ui_dst)
    deg_uu = _make_degrees(NK_UU)(uu_src, uu_dst)
    sc_ui = scales(deg_ui.reshape(NW, 2, NP, 1))
    sc_uu = scales(deg_uu.reshape(NW, 2, NP, 1))

    agg = _make_aggregate()
    mm_ui = _make_matmul_ui()
    mm_uu = _make_matmul_uu()
    post_ui = _make_post(N_UI, 10000)
    post_uu = _make_post(N_UU, 5000)

    emb_ui0, carry_ui0 = _make_init_ui()(user_embeddings, item_embeddings)
    emb_uu0 = _make_init_uu()(user_embeddings)

    def layer(c, ws):
        emb_ui, carry_ui, emb_uu, carry_uu = c
        wu, wv, wuu = ws
        nodef_ui = mm_ui(emb_ui, wu, wv, sc_ui)
        nodef_uu = mm_uu(emb_uu, wuu, sc_uu)
        parts_ui, parts_uu = agg(nodef_ui, ui_src, ui_dst,
                                 nodef_uu, uu_src, uu_dst)
        emb_ui, carry_ui = post_ui(parts_ui, parts_ui, sc_ui, carry_ui)
        emb_uu, carry_uu = post_uu(parts_uu, parts_uu, sc_uu, carry_uu)
        return (emb_ui, carry_ui, emb_uu, carry_uu), None

    c0 = (emb_ui0, carry_ui0, emb_uu0, user_embeddings)
    (_, carry_ui, _, carry_uu), _ = lax.scan(
        layer, c0, (ui_u_w, ui_v_w, uu_u_w))

    return carry_ui, carry_uu
